# Initial kernel scaffold; baseline (speedup 1.0000x reference)
import functools

import jax
import jax.numpy as jnp
from jax import lax
from jax.experimental import pallas as pl
from jax.experimental.pallas import tpu as pltpu
from jax.experimental.pallas import tpu_sc as plsc

VOCAB = 100000
DIM = 64
BATCH = 4096
SEQ = 200
B = BATCH * SEQ
NC = 2
NS = 16
NW = NC * NS
B_PER_W = B // NW
CHUNK = 1024
NCHUNK = B_PER_W // CHUNK


@functools.partial(
    pl.kernel,
    mesh=plsc.VectorSubcoreMesh(core_axis_name="c", subcore_axis_name="s"),
    out_type=jax.ShapeDtypeStruct((B, DIM), jnp.float32),
    scratch_types=[
        pltpu.VMEM((CHUNK,), jnp.int32),
        pltpu.VMEM((CHUNK, DIM), jnp.float32),
        pltpu.SemaphoreType.DMA,
    ],
)
def _gather_kernel(idx_hbm, table_hbm, out_hbm, idx_v, rows_v, sem):
    wid = lax.axis_index("s") * NC + lax.axis_index("c")
    base = wid * B_PER_W

    def body(i, _):
        off = base + i * CHUNK
        pltpu.sync_copy(idx_hbm.at[pl.ds(off, CHUNK)], idx_v)
        pltpu.async_copy(table_hbm.at[idx_v], rows_v, sem).wait()
        pltpu.sync_copy(rows_v, out_hbm.at[pl.ds(off, CHUNK)])
        return ()

    lax.fori_loop(0, NCHUNK, body, ())


def kernel(x, table):
    flat = _gather_kernel(x.reshape(B), table)
    return flat.reshape(BATCH, SEQ, DIM)


# speedup vs baseline: 4.1379x; 4.1379x over previous
import functools

import jax
import jax.numpy as jnp
from jax import lax
from jax.experimental import pallas as pl
from jax.experimental.pallas import tpu as pltpu
from jax.experimental.pallas import tpu_sc as plsc

VOCAB = 100000
DIM = 64
BATCH = 4096
SEQ = 200
B = BATCH * SEQ
NC = 2
NS = 16
NW = NC * NS
B_PER_W = B // NW
CHUNK = 1024
NCHUNK = B_PER_W // CHUNK


@functools.partial(
    pl.kernel,
    mesh=plsc.VectorSubcoreMesh(core_axis_name="c", subcore_axis_name="s"),
    out_type=jax.ShapeDtypeStruct((B, DIM), jnp.float32),
    scratch_types=[
        pltpu.VMEM((CHUNK,), jnp.int32),
        pltpu.VMEM((CHUNK, DIM), jnp.float32),
        pltpu.SemaphoreType.DMA,
    ],
    compiler_params=pltpu.CompilerParams(use_tc_tiling_on_sc=False),
)
def _gather_kernel(idx_hbm, table_hbm, out_hbm, idx_v, rows_v, sem):
    wid = lax.axis_index("s") * NC + lax.axis_index("c")
    base = wid * B_PER_W

    def body(i, _):
        off = base + i * CHUNK
        pltpu.sync_copy(idx_hbm.at[pl.ds(off, CHUNK)], idx_v)
        pltpu.async_copy(table_hbm.at[idx_v], rows_v, sem).wait()
        pltpu.sync_copy(rows_v, out_hbm.at[pl.ds(off, CHUNK)])
        return ()

    lax.fori_loop(0, NCHUNK, body, ())


def kernel(x, table):
    flat = _gather_kernel(x.reshape(B), table)
    return flat.reshape(BATCH, SEQ, DIM)


# speedup vs baseline: 4.2615x; 1.0299x over previous
import functools

import jax
import jax.numpy as jnp
from jax import lax
from jax.experimental import pallas as pl
from jax.experimental.pallas import tpu as pltpu
from jax.experimental.pallas import tpu_sc as plsc

VOCAB = 100000
DIM = 64
BATCH = 4096
SEQ = 200
B = BATCH * SEQ
NC = 2
NS = 16
NW = NC * NS
B_PER_W = B // NW
CHUNK = 800
NCHUNK = B_PER_W // CHUNK
NPAIR = NCHUNK // 2


@functools.partial(
    pl.kernel,
    mesh=plsc.VectorSubcoreMesh(core_axis_name="c", subcore_axis_name="s"),
    out_type=jax.ShapeDtypeStruct((B, DIM), jnp.float32),
    scratch_types=[
        pltpu.VMEM((CHUNK,), jnp.int32),
        pltpu.VMEM((CHUNK,), jnp.int32),
        pltpu.VMEM((CHUNK, DIM), jnp.float32),
        pltpu.VMEM((CHUNK, DIM), jnp.float32),
        pltpu.SemaphoreType.DMA,
        pltpu.SemaphoreType.DMA,
        pltpu.SemaphoreType.DMA,
        pltpu.SemaphoreType.DMA,
    ],
    compiler_params=pltpu.CompilerParams(use_tc_tiling_on_sc=False),
)
def _gather_kernel(idx_hbm, table_hbm, out_hbm, idx0, idx1, rows0, rows1,
                   sg0, sg1, ss0, ss1):
    wid = lax.axis_index("s") * NC + lax.axis_index("c")
    base = wid * B_PER_W

    def gather_wait(idx_v, rows_v, sem):
        pltpu.make_async_copy(table_hbm.at[idx_v], rows_v, sem).wait()

    def store_wait(rows_v, off, sem):
        pltpu.make_async_copy(rows_v, out_hbm.at[pl.ds(off, CHUNK)], sem).wait()

    pltpu.sync_copy(idx_hbm.at[pl.ds(base, CHUNK)], idx0)
    pltpu.async_copy(table_hbm.at[idx0], rows0, sg0)

    def body(j, _):
        off0 = base + (2 * j) * CHUNK
        off1 = off0 + CHUNK

        pltpu.sync_copy(idx_hbm.at[pl.ds(off1, CHUNK)], idx1)

        @pl.when(j > 0)
        def _():
            store_wait(rows1, off1 - 2 * CHUNK, ss1)

        pltpu.async_copy(table_hbm.at[idx1], rows1, sg1)

        gather_wait(idx0, rows0, sg0)
        pltpu.async_copy(rows0, out_hbm.at[pl.ds(off0, CHUNK)], ss0)

        @pl.when(j < NPAIR - 1)
        def _():
            pltpu.sync_copy(idx_hbm.at[pl.ds(off0 + 2 * CHUNK, CHUNK)], idx0)
            store_wait(rows0, off0, ss0)
            pltpu.async_copy(table_hbm.at[idx0], rows0, sg0)

        gather_wait(idx1, rows1, sg1)
        pltpu.async_copy(rows1, out_hbm.at[pl.ds(off1, CHUNK)], ss1)
        return ()

    lax.fori_loop(0, NPAIR, body, ())

    last0 = base + (NCHUNK - 2) * CHUNK
    store_wait(rows0, last0, ss0)
    store_wait(rows1, last0 + CHUNK, ss1)


def kernel(x, table):
    flat = _gather_kernel(x.reshape(B), table)
    return flat.reshape(BATCH, SEQ, DIM)


# speedup vs baseline: 4.2628x; 1.0003x over previous
import functools

import jax
import jax.numpy as jnp
from jax import lax
from jax.experimental import pallas as pl
from jax.experimental.pallas import tpu as pltpu
from jax.experimental.pallas import tpu_sc as plsc

VOCAB = 100000
DIM = 64
BATCH = 4096
SEQ = 200
B = BATCH * SEQ
NC = 2
NS = 16
NW = NC * NS
B_PER_W = B // NW
NBUF = 4
CHUNK = 400
NCHUNK = B_PER_W // CHUNK
NROUND = NCHUNK // NBUF


@functools.partial(
    pl.kernel,
    mesh=plsc.VectorSubcoreMesh(core_axis_name="c", subcore_axis_name="s"),
    out_type=jax.ShapeDtypeStruct((B, DIM), jnp.float32),
    scratch_types=(
        [pltpu.VMEM((CHUNK,), jnp.int32) for _ in range(NBUF)]
        + [pltpu.VMEM((CHUNK, DIM), jnp.float32) for _ in range(NBUF)]
        + [pltpu.SemaphoreType.DMA for _ in range(2 * NBUF)]
    ),
    compiler_params=pltpu.CompilerParams(use_tc_tiling_on_sc=False),
)
def _gather_kernel(idx_hbm, table_hbm, out_hbm, *scratch):
    idx = scratch[:NBUF]
    rows = scratch[NBUF:2 * NBUF]
    sg = scratch[2 * NBUF:3 * NBUF]
    ss = scratch[3 * NBUF:4 * NBUF]

    wid = lax.axis_index("s") * NC + lax.axis_index("c")
    base = wid * B_PER_W

    for b in range(NBUF):
        pltpu.sync_copy(idx_hbm.at[pl.ds(base + b * CHUNK, CHUNK)], idx[b])
        pltpu.async_copy(table_hbm.at[idx[b]], rows[b], sg[b])

    def body(r, _):
        off = base + r * NBUF * CHUNK
        for b in range(NBUF):
            goff = off + b * CHUNK
            pltpu.make_async_copy(table_hbm.at[idx[b]], rows[b], sg[b]).wait()
            pltpu.async_copy(rows[b], out_hbm.at[pl.ds(goff, CHUNK)], ss[b])

            @pl.when(r < NROUND - 1)
            def _():
                noff = goff + NBUF * CHUNK
                pltpu.sync_copy(idx_hbm.at[pl.ds(noff, CHUNK)], idx[b])
                pltpu.make_async_copy(
                    rows[b], out_hbm.at[pl.ds(goff, CHUNK)], ss[b]).wait()
                pltpu.async_copy(table_hbm.at[idx[b]], rows[b], sg[b])
        return ()

    lax.fori_loop(0, NROUND, body, ())

    last = base + (NROUND - 1) * NBUF * CHUNK
    for b in range(NBUF):
        pltpu.make_async_copy(
            rows[b], out_hbm.at[pl.ds(last + b * CHUNK, CHUNK)], ss[b]).wait()


def kernel(x, table):
    flat = _gather_kernel(x.reshape(B), table)
    return flat.reshape(BATCH, SEQ, DIM)
